# baseline (speedup 1.0000x reference)
import dataclasses
import functools

import jax
import jax.numpy as jnp
from jax import lax
from jax.experimental import pallas as pl
from jax.experimental.pallas import tpu as pltpu
from jax.experimental.pallas import tpu_sc as plsc

N = 10000
E = 160000
D_IN = 256
D_OUT = 256
H = 128
NC = 2
NS = 16
EPT = E // NS
EP = 163840
EPT2 = EP // NS
CH = 128
IP = 5
NCHUNK = EPT2 // CH
CPP = NCHUNK // IP
NPA = 10112
RPTA = NPA // NS
NP = 10240
RPT = NP // NS
EPW = EPT // 16
W5 = RPT // 128

_mesh = plsc.VectorSubcoreMesh(core_axis_name="c", subcore_axis_name="s")


@functools.partial(
    pl.kernel,
    out_type=jax.ShapeDtypeStruct((NC, NS, W5, 128), jnp.float32),
    mesh=_mesh,
    scratch_types=[
        pltpu.VMEM((EPW, 16), jnp.int32),
        pltpu.VMEM((NP,), jnp.float32),
        pltpu.VMEM((NS, RPT), jnp.float32),
        pltpu.VMEM((W5, 128), jnp.float32),
        pltpu.VMEM_SHARED((NS, NP), jnp.float32),
        pltpu.SemaphoreType.DMA,
    ],
    compiler_params=dataclasses.replace(pltpu.CompilerParams(),
                                        needs_layout_passes=False),
)
def _deg_kernel(srcw, dstw, deg_out,
                idx_v, hist_v, merge_v, wide_v, hist_sh, sem):
    c = lax.axis_index("c")
    s = lax.axis_index("s")

    @pl.loop(0, NP // 16)
    def _(i):
        hist_v[pl.ds(i * 16, 16)] = jnp.zeros((16,), jnp.float32)

    @pl.when(c == 0)
    def _():
        pltpu.sync_copy(srcw.at[s], idx_v)

    @pl.when(c == 1)
    def _():
        pltpu.sync_copy(dstw.at[s], idx_v)

    @pl.loop(0, EPW)
    def _(j):
        idx16 = idx_v[j, :]
        cnt, last = plsc.scan_count(idx16)
        plsc.addupdate_scatter(hist_v, [idx16],
                               cnt.astype(jnp.float32), mask=last)

    pltpu.sync_copy(hist_v, hist_sh.at[s])
    plsc.subcore_barrier()
    pltpu.sync_copy(hist_sh.at[:, pl.ds(s * RPT, RPT)], merge_v)

    @pl.loop(0, RPT // 16)
    def _(k):
        def body(r, a):
            return a + merge_v[r, pl.ds(k * 16, 16)]
        acc = lax.fori_loop(0, NS, body, jnp.zeros((16,), jnp.float32))
        wide_v[k // 8, pl.ds((k % 8) * 16, 16)] = acc

    pltpu.sync_copy(wide_v, deg_out.at[c, s])


@functools.partial(
    pl.kernel,
    out_type=jax.ShapeDtypeStruct((NC, NPA, H), jnp.float32),
    mesh=_mesh,
    scratch_types=[
        pltpu.VMEM((CPP, CH), jnp.int32),
        pltpu.VMEM((CPP, CH), jnp.int32),
        pltpu.VMEM((CH, H), jnp.float32),
        pltpu.VMEM((CH, H), jnp.float32),
        pltpu.VMEM_SHARED((NPA, H), jnp.float32),
        pltpu.SemaphoreType.DMA,
        pltpu.SemaphoreType.DMA,
    ],
)
def _agg_kernel(h2, src3, src3p, dst3, z_hbm, agg_out,
                sidx_v, didx_v, rows_a, rows_b, acc_sh, sem_a, sem_b):
    c = lax.axis_index("c")
    s = lax.axis_index("s")
    pltpu.sync_copy(z_hbm, acc_sh.at[pl.ds(s * RPTA, RPTA)])
    plsc.subcore_barrier()

    @pl.loop(0, IP)
    def _(ph):
        @pl.when(c == 0)
        def _():
            pltpu.sync_copy(src3.at[s, ph], sidx_v)

        @pl.when(c == 1)
        def _():
            pltpu.sync_copy(src3p.at[s, ph], sidx_v)

        pltpu.sync_copy(dst3.at[s, ph], didx_v)

        pltpu.async_copy(h2.at[sidx_v.at[0]], rows_a, sem_a)

        @pl.loop(0, CPP // 2)
        def _(p):
            j0 = 2 * p
            pltpu.async_copy(h2.at[sidx_v.at[j0 + 1]], rows_b, sem_b)
            pltpu.make_async_copy(h2.at[sidx_v.at[j0]], rows_a, sem_a).wait()
            pltpu.sync_copy(rows_a, acc_sh.at[didx_v.at[j0]], add=True)

            @pl.when(p < CPP // 2 - 1)
            def _():
                pltpu.async_copy(h2.at[sidx_v.at[j0 + 2]], rows_a, sem_a)

            pltpu.make_async_copy(h2.at[sidx_v.at[j0 + 1]], rows_b, sem_b).wait()
            pltpu.sync_copy(rows_b, acc_sh.at[didx_v.at[j0 + 1]], add=True)

    plsc.subcore_barrier()
    pltpu.sync_copy(acc_sh.at[pl.ds(s * RPTA, RPTA)],
                    agg_out.at[c, pl.ds(s * RPTA, RPTA)])


def _mm_body(feat_ref, w_ref, ti_ref, wt_ref, out_ref):
    t = ti_ref[...]
    ws = jnp.where(t == 0, wt_ref[0],
                   jnp.where(t == 1, wt_ref[1],
                             jnp.where(t == 2, wt_ref[2], wt_ref[3])))
    x = feat_ref[...] * ws
    h = jnp.dot(x, w_ref[...], preferred_element_type=jnp.float32)
    out_ref[0] = h[:, :H]
    out_ref[1] = h[:, H:]


def _scale_body(h0_ref, deg_ref, out_ref):
    nl = lax.rsqrt(jnp.maximum(deg_ref[0], 1.0))
    out_ref[0] = h0_ref[0] * nl
    out_ref[1] = h0_ref[1] * nl


def _fin_body(agg_ref, deg_ref, bias_ref, out_ref):
    nr = lax.rsqrt(jnp.maximum(deg_ref[0], 1.0))
    h = jnp.concatenate([agg_ref[0], agg_ref[1]], axis=1)
    out_ref[...] = h * nr + bias_ref[...]


_R = 2000


def kernel(feat, edge_index, type_info, weight, bias, weight_type):
    src = edge_index[0]
    dst = edge_index[1]
    npad = EP - E
    pad_i = jnp.arange(npad, dtype=jnp.int32)
    src_p = jnp.concatenate([src, pad_i % N])
    dst_p = jnp.concatenate([dst, N + pad_i % (NPA - N)])
    src3 = src_p.reshape(NS, IP, CPP, CH)
    dst3 = dst_p.reshape(NS, IP, CPP, CH)
    src3p = src3 + N
    z128 = jnp.zeros((RPTA, H), jnp.float32)

    degw = _deg_kernel(src.reshape(NS, EPW, 16), dst.reshape(NS, EPW, 16))
    deg = degw.reshape(NC, NP)[:, :N].reshape(NC, N, 1)

    ti = type_info.reshape(N, 1)
    h0 = pl.pallas_call(
        _mm_body,
        grid=(N // _R,),
        in_specs=[
            pl.BlockSpec((_R, D_IN), lambda i: (i, 0)),
            pl.BlockSpec((D_IN, D_OUT), lambda i: (0, 0)),
            pl.BlockSpec((_R, 1), lambda i: (i, 0)),
            pl.BlockSpec(memory_space=pltpu.SMEM),
        ],
        out_specs=pl.BlockSpec((NC, _R, H), lambda i: (0, i, 0)),
        out_shape=jax.ShapeDtypeStruct((NC, N, H), jnp.float32),
    )(feat, weight, ti, weight_type)

    h2 = pl.pallas_call(
        _scale_body,
        grid=(N // _R,),
        in_specs=[
            pl.BlockSpec((NC, _R, H), lambda i: (0, i, 0)),
            pl.BlockSpec((1, _R, 1), lambda i: (0, i, 0)),
        ],
        out_specs=pl.BlockSpec((NC, _R, H), lambda i: (0, i, 0)),
        out_shape=jax.ShapeDtypeStruct((NC, N, H), jnp.float32),
    )(h0, deg)

    agg2 = _agg_kernel(h2.reshape(NC * N, H), src3, src3p, dst3, z128)

    out = pl.pallas_call(
        _fin_body,
        grid=(N // _R,),
        in_specs=[
            pl.BlockSpec((NC, _R, H), lambda i: (0, i, 0)),
            pl.BlockSpec((1, _R, 1), lambda i: (1, i, 0)),
            pl.BlockSpec((1, D_OUT), lambda i: (0, 0)),
        ],
        out_specs=pl.BlockSpec((_R, D_OUT), lambda i: (i, 0)),
        out_shape=jax.ShapeDtypeStruct((N, D_OUT), jnp.float32),
    )(agg2, deg, bias.reshape(1, D_OUT))
    return out


# speedup vs baseline: 1.0244x; 1.0244x over previous
import dataclasses
import functools

import jax
import jax.numpy as jnp
from jax import lax
from jax.experimental import pallas as pl
from jax.experimental.pallas import tpu as pltpu
from jax.experimental.pallas import tpu_sc as plsc

N = 10000
E = 160000
D_IN = 256
D_OUT = 256
H = 128
NC = 2
NS = 16
EPT = E // NS
CH = 100
IP = 2
NCHUNK = EPT // CH
CPP = NCHUNK // IP
NP = 10240
RPT = NP // NS
EPW = EPT // 16
W5 = RPT // 128

_mesh = plsc.VectorSubcoreMesh(core_axis_name="c", subcore_axis_name="s")


@functools.partial(
    pl.kernel,
    out_type=jax.ShapeDtypeStruct((NC, NS, W5, 128), jnp.float32),
    mesh=_mesh,
    scratch_types=[
        pltpu.VMEM((EPW, 16), jnp.int32),
        pltpu.VMEM((NP,), jnp.float32),
        pltpu.VMEM((NS, RPT), jnp.float32),
        pltpu.VMEM((W5, 128), jnp.float32),
        pltpu.VMEM_SHARED((NS, NP), jnp.float32),
        pltpu.SemaphoreType.DMA,
    ],
    compiler_params=dataclasses.replace(pltpu.CompilerParams(),
                                        needs_layout_passes=False),
)
def _deg_kernel(srcw, dstw, deg_out,
                idx_v, hist_v, merge_v, wide_v, hist_sh, sem):
    c = lax.axis_index("c")
    s = lax.axis_index("s")

    @pl.loop(0, NP // 16)
    def _(i):
        hist_v[pl.ds(i * 16, 16)] = jnp.zeros((16,), jnp.float32)

    @pl.when(c == 0)
    def _():
        pltpu.sync_copy(srcw.at[s], idx_v)

    @pl.when(c == 1)
    def _():
        pltpu.sync_copy(dstw.at[s], idx_v)

    @pl.loop(0, EPW)
    def _(j):
        idx16 = idx_v[j, :]
        cnt, last = plsc.scan_count(idx16)
        plsc.addupdate_scatter(hist_v, [idx16],
                               cnt.astype(jnp.float32), mask=last)

    pltpu.sync_copy(hist_v, hist_sh.at[s])
    plsc.subcore_barrier()
    pltpu.sync_copy(hist_sh.at[:, pl.ds(s * RPT, RPT)], merge_v)

    @pl.loop(0, RPT // 16)
    def _(k):
        def body(r, a):
            return a + merge_v[r, pl.ds(k * 16, 16)]
        acc = lax.fori_loop(0, NS, body, jnp.zeros((16,), jnp.float32))
        wide_v[k // 8, pl.ds((k % 8) * 16, 16)] = acc

    pltpu.sync_copy(wide_v, deg_out.at[c, s])


@functools.partial(
    pl.kernel,
    out_type=jax.ShapeDtypeStruct((NC, NP, H), jnp.float32),
    mesh=_mesh,
    scratch_types=[
        pltpu.VMEM((CPP, CH), jnp.int32),
        pltpu.VMEM((CPP, CH), jnp.int32),
        pltpu.VMEM((CH, H), jnp.float32),
        pltpu.VMEM((CH, H), jnp.float32),
        pltpu.VMEM_SHARED((NP, H), jnp.float32),
        pltpu.SemaphoreType.DMA,
        pltpu.SemaphoreType.DMA,
    ],
)
def _agg_kernel(h2, src3, src3p, dst3, z_hbm, agg_out,
                sidx_v, didx_v, rows_a, rows_b, acc_sh, sem_a, sem_b):
    c = lax.axis_index("c")
    s = lax.axis_index("s")
    pltpu.sync_copy(z_hbm, acc_sh.at[pl.ds(s * RPT, RPT)])
    plsc.subcore_barrier()

    @pl.loop(0, IP)
    def _(ph):
        @pl.when(c == 0)
        def _():
            pltpu.sync_copy(src3.at[s, ph], sidx_v)

        @pl.when(c == 1)
        def _():
            pltpu.sync_copy(src3p.at[s, ph], sidx_v)

        pltpu.sync_copy(dst3.at[s, ph], didx_v)

        pltpu.async_copy(h2.at[sidx_v.at[0]], rows_a, sem_a)

        @pl.loop(0, CPP // 2)
        def _(p):
            j0 = 2 * p
            pltpu.async_copy(h2.at[sidx_v.at[j0 + 1]], rows_b, sem_b)
            pltpu.make_async_copy(h2.at[sidx_v.at[j0]], rows_a, sem_a).wait()
            pltpu.sync_copy(rows_a, acc_sh.at[didx_v.at[j0]], add=True)

            @pl.when(p < CPP // 2 - 1)
            def _():
                pltpu.async_copy(h2.at[sidx_v.at[j0 + 2]], rows_a, sem_a)

            pltpu.make_async_copy(h2.at[sidx_v.at[j0 + 1]], rows_b, sem_b).wait()
            pltpu.sync_copy(rows_b, acc_sh.at[didx_v.at[j0 + 1]], add=True)

    plsc.subcore_barrier()
    pltpu.sync_copy(acc_sh.at[pl.ds(s * RPT, RPT)],
                    agg_out.at[c, pl.ds(s * RPT, RPT)])


def _mm_body(feat_ref, w_ref, ti_ref, wt_ref, out_ref):
    t = ti_ref[...]
    ws = jnp.where(t == 0, wt_ref[0],
                   jnp.where(t == 1, wt_ref[1],
                             jnp.where(t == 2, wt_ref[2], wt_ref[3])))
    x = feat_ref[...] * ws
    h = jnp.dot(x, w_ref[...], preferred_element_type=jnp.float32)
    out_ref[0] = h[:, :H]
    out_ref[1] = h[:, H:]


def _scale_body(h0_ref, deg_ref, out_ref):
    nl = lax.rsqrt(jnp.maximum(deg_ref[0], 1.0))
    out_ref[0] = h0_ref[0] * nl
    out_ref[1] = h0_ref[1] * nl


def _fin_body(agg_ref, deg_ref, bias_ref, out_ref):
    nr = lax.rsqrt(jnp.maximum(deg_ref[0], 1.0))
    h = jnp.concatenate([agg_ref[0], agg_ref[1]], axis=1)
    out_ref[...] = h * nr + bias_ref[...]


_R = 2000


def kernel(feat, edge_index, type_info, weight, bias, weight_type):
    src = edge_index[0]
    dst = edge_index[1]
    src3 = src.reshape(NS, NCHUNK, CH)
    dst3 = dst.reshape(NS, NCHUNK, CH)
    src3p = src3 + N
    z128 = jnp.zeros((RPT, H), jnp.float32)

    degw = _deg_kernel(src.reshape(NS, EPW, 16), dst.reshape(NS, EPW, 16))
    deg = degw.reshape(NC, NP)[:, :N].reshape(NC, N, 1)

    ti = type_info.reshape(N, 1)
    h0 = pl.pallas_call(
        _mm_body,
        grid=(N // _R,),
        in_specs=[
            pl.BlockSpec((_R, D_IN), lambda i: (i, 0)),
            pl.BlockSpec((D_IN, D_OUT), lambda i: (0, 0)),
            pl.BlockSpec((_R, 1), lambda i: (i, 0)),
            pl.BlockSpec(memory_space=pltpu.SMEM),
        ],
        out_specs=pl.BlockSpec((NC, _R, H), lambda i: (0, i, 0)),
        out_shape=jax.ShapeDtypeStruct((NC, N, H), jnp.float32),
    )(feat, weight, ti, weight_type)

    h2 = pl.pallas_call(
        _scale_body,
        grid=(N // _R,),
        in_specs=[
            pl.BlockSpec((NC, _R, H), lambda i: (0, i, 0)),
            pl.BlockSpec((1, _R, 1), lambda i: (0, i, 0)),
        ],
        out_specs=pl.BlockSpec((NC, _R, H), lambda i: (0, i, 0)),
        out_shape=jax.ShapeDtypeStruct((NC, N, H), jnp.float32),
    )(h0, deg)

    agg2 = _agg_kernel(h2.reshape(NC * N, H),
                       src3.reshape(NS, IP, CPP, CH),
                       src3p.reshape(NS, IP, CPP, CH),
                       dst3.reshape(NS, IP, CPP, CH), z128)

    out = pl.pallas_call(
        _fin_body,
        grid=(N // _R,),
        in_specs=[
            pl.BlockSpec((NC, _R, H), lambda i: (0, i, 0)),
            pl.BlockSpec((1, _R, 1), lambda i: (1, i, 0)),
            pl.BlockSpec((1, D_OUT), lambda i: (0, 0)),
        ],
        out_specs=pl.BlockSpec((_R, D_OUT), lambda i: (i, 0)),
        out_shape=jax.ShapeDtypeStruct((N, D_OUT), jnp.float32),
    )(agg2, deg, bias.reshape(1, D_OUT))
    return out


# speedup vs baseline: 1.0468x; 1.0218x over previous
import dataclasses
import functools

import jax
import jax.numpy as jnp
from jax import lax
from jax.experimental import pallas as pl
from jax.experimental.pallas import tpu as pltpu
from jax.experimental.pallas import tpu_sc as plsc

N = 10000
E = 160000
D_IN = 256
D_OUT = 256
H = 128
NC = 2
NS = 16
EPT = E // NS
CH = 100
IP = 2
NCHUNK = EPT // CH
CPP = NCHUNK // IP
NP = 10240
RPT = NP // NS
EPW = EPT // 16
W5 = RPT // 128

_mesh = plsc.VectorSubcoreMesh(core_axis_name="c", subcore_axis_name="s")


@functools.partial(
    pl.kernel,
    out_type=jax.ShapeDtypeStruct((NC, NS, W5, 128), jnp.float32),
    mesh=_mesh,
    scratch_types=[
        pltpu.VMEM((EPW, 16), jnp.int32),
        pltpu.VMEM((NP,), jnp.float32),
        pltpu.VMEM((NS, RPT), jnp.float32),
        pltpu.VMEM((W5, 128), jnp.float32),
        pltpu.VMEM_SHARED((NS, NP), jnp.float32),
        pltpu.SemaphoreType.DMA,
    ],
    compiler_params=dataclasses.replace(pltpu.CompilerParams(),
                                        needs_layout_passes=False),
)
def _deg_kernel(srcw, dstw, deg_out,
                idx_v, hist_v, merge_v, wide_v, hist_sh, sem):
    c = lax.axis_index("c")
    s = lax.axis_index("s")

    @pl.loop(0, NP // 16)
    def _(i):
        hist_v[pl.ds(i * 16, 16)] = jnp.zeros((16,), jnp.float32)

    @pl.when(c == 0)
    def _():
        pltpu.sync_copy(srcw.at[s], idx_v)

    @pl.when(c == 1)
    def _():
        pltpu.sync_copy(dstw.at[s], idx_v)

    @pl.loop(0, EPW)
    def _(j):
        idx16 = idx_v[j, :]
        cnt, last = plsc.scan_count(idx16)
        plsc.addupdate_scatter(hist_v, [idx16],
                               cnt.astype(jnp.float32), mask=last)

    pltpu.sync_copy(hist_v, hist_sh.at[s])
    plsc.subcore_barrier()
    pltpu.sync_copy(hist_sh.at[:, pl.ds(s * RPT, RPT)], merge_v)

    @pl.loop(0, RPT // 16)
    def _(k):
        def body(r, a):
            return a + merge_v[r, pl.ds(k * 16, 16)]
        acc = lax.fori_loop(0, NS, body, jnp.zeros((16,), jnp.float32))
        wide_v[k // 8, pl.ds((k % 8) * 16, 16)] = acc

    pltpu.sync_copy(wide_v, deg_out.at[c, s])


@functools.partial(
    pl.kernel,
    out_type=jax.ShapeDtypeStruct((NC, NP, H), jnp.float32),
    mesh=_mesh,
    scratch_types=[
        pltpu.VMEM((CPP, CH), jnp.int32),
        pltpu.VMEM((CPP, CH), jnp.int32),
        pltpu.VMEM((CH, H), jnp.float32),
        pltpu.VMEM((CH, H), jnp.float32),
        pltpu.VMEM_SHARED((NP, H), jnp.float32),
        pltpu.SemaphoreType.DMA,
        pltpu.SemaphoreType.DMA,
    ],
)
def _agg_kernel(h2, src3, src3p, dst3, agg_out,
                sidx_v, didx_v, rows_a, rows_b, acc_sh, sem_a, sem_b):
    c = lax.axis_index("c")
    s = lax.axis_index("s")

    @pl.loop(0, CH)
    def _(i):
        @pl.loop(0, H // 16)
        def _(k):
            rows_a[i, pl.ds(k * 16, 16)] = jnp.zeros((16,), jnp.float32)

    @pl.loop(0, RPT // 80)
    def _(k):
        pltpu.sync_copy(rows_a.at[pl.ds(0, 80)],
                        acc_sh.at[pl.ds(s * RPT + k * 80, 80)])

    plsc.subcore_barrier()

    @pl.loop(0, IP)
    def _(ph):
        @pl.when(c == 0)
        def _():
            pltpu.sync_copy(src3.at[s, ph], sidx_v)

        @pl.when(c == 1)
        def _():
            pltpu.sync_copy(src3p.at[s, ph], sidx_v)

        pltpu.sync_copy(dst3.at[s, ph], didx_v)

        pltpu.async_copy(h2.at[sidx_v.at[0]], rows_a, sem_a)

        @pl.loop(0, CPP // 2)
        def _(p):
            j0 = 2 * p
            pltpu.async_copy(h2.at[sidx_v.at[j0 + 1]], rows_b, sem_b)
            pltpu.make_async_copy(h2.at[sidx_v.at[j0]], rows_a, sem_a).wait()
            pltpu.sync_copy(rows_a, acc_sh.at[didx_v.at[j0]], add=True)

            @pl.when(p < CPP // 2 - 1)
            def _():
                pltpu.async_copy(h2.at[sidx_v.at[j0 + 2]], rows_a, sem_a)

            pltpu.make_async_copy(h2.at[sidx_v.at[j0 + 1]], rows_b, sem_b).wait()
            pltpu.sync_copy(rows_b, acc_sh.at[didx_v.at[j0 + 1]], add=True)

    plsc.subcore_barrier()
    pltpu.sync_copy(acc_sh.at[pl.ds(s * RPT, RPT)],
                    agg_out.at[c, pl.ds(s * RPT, RPT)])


def _mm_body(feat_ref, w_ref, ti_ref, wt_ref, out_ref):
    t = ti_ref[...]
    ws = jnp.where(t == 0, wt_ref[0],
                   jnp.where(t == 1, wt_ref[1],
                             jnp.where(t == 2, wt_ref[2], wt_ref[3])))
    x = feat_ref[...] * ws
    h = jnp.dot(x, w_ref[...], preferred_element_type=jnp.float32)
    out_ref[0] = h[:, :H]
    out_ref[1] = h[:, H:]


def _scale_body(h0_ref, deg_ref, out_ref):
    nl = lax.rsqrt(jnp.maximum(deg_ref[0], 1.0))
    out_ref[0] = h0_ref[0] * nl
    out_ref[1] = h0_ref[1] * nl


def _fin_body(agg_ref, deg_ref, bias_ref, out_ref):
    nr = lax.rsqrt(jnp.maximum(deg_ref[0], 1.0))
    h = jnp.concatenate([agg_ref[0], agg_ref[1]], axis=1)
    out_ref[...] = h * nr + bias_ref[...]


_R = 2000


def kernel(feat, edge_index, type_info, weight, bias, weight_type):
    src = edge_index[0]
    dst = edge_index[1]
    src3 = src.reshape(NS, NCHUNK, CH)
    dst3 = dst.reshape(NS, NCHUNK, CH)
    src3p = src3 + N

    degw = _deg_kernel(src.reshape(NS, EPW, 16), dst.reshape(NS, EPW, 16))
    deg = degw.reshape(NC, NP)[:, :N].reshape(NC, N, 1)

    ti = type_info.reshape(N, 1)
    h0 = pl.pallas_call(
        _mm_body,
        grid=(N // _R,),
        in_specs=[
            pl.BlockSpec((_R, D_IN), lambda i: (i, 0)),
            pl.BlockSpec((D_IN, D_OUT), lambda i: (0, 0)),
            pl.BlockSpec((_R, 1), lambda i: (i, 0)),
            pl.BlockSpec(memory_space=pltpu.SMEM),
        ],
        out_specs=pl.BlockSpec((NC, _R, H), lambda i: (0, i, 0)),
        out_shape=jax.ShapeDtypeStruct((NC, N, H), jnp.float32),
    )(feat, weight, ti, weight_type)

    h2 = pl.pallas_call(
        _scale_body,
        grid=(N // _R,),
        in_specs=[
            pl.BlockSpec((NC, _R, H), lambda i: (0, i, 0)),
            pl.BlockSpec((1, _R, 1), lambda i: (0, i, 0)),
        ],
        out_specs=pl.BlockSpec((NC, _R, H), lambda i: (0, i, 0)),
        out_shape=jax.ShapeDtypeStruct((NC, N, H), jnp.float32),
    )(h0, deg)

    agg2 = _agg_kernel(h2.reshape(NC * N, H),
                       src3.reshape(NS, IP, CPP, CH),
                       src3p.reshape(NS, IP, CPP, CH),
                       dst3.reshape(NS, IP, CPP, CH))

    out = pl.pallas_call(
        _fin_body,
        grid=(N // _R,),
        in_specs=[
            pl.BlockSpec((NC, _R, H), lambda i: (0, i, 0)),
            pl.BlockSpec((1, _R, 1), lambda i: (1, i, 0)),
            pl.BlockSpec((1, D_OUT), lambda i: (0, 0)),
        ],
        out_specs=pl.BlockSpec((_R, D_OUT), lambda i: (i, 0)),
        out_shape=jax.ShapeDtypeStruct((N, D_OUT), jnp.float32),
    )(agg2, deg, bias.reshape(1, D_OUT))
    return out
